# baseline (speedup 1.0000x reference)
import functools
import jax
import jax.numpy as jnp
from jax import lax
from jax.experimental import pallas as pl
from jax.experimental.pallas import tpu as pltpu
from jax.experimental.pallas import tpu_sc as plsc

_WIN = 32
_K = 8
_NBUF = 4
_CHUNK = 16


def _modavg_kernel(vis_ref, q_ref, w1_ref, w2_ref, ma_ref):
    T = vis_ref.shape[1]
    C = vis_ref.shape[2]
    L = q_ref.shape[1]
    M = T - 3

    vis = vis_ref[0]
    q = q_ref[0]
    e1 = jnp.dot(q, w1_ref[...], preferred_element_type=jnp.float32)
    e2 = jnp.dot(q, w2_ref[...], preferred_element_type=jnp.float32)
    sim = jax.lax.dot_general(
        vis, e1, (((1,), (1,)), ((), ())),
        preferred_element_type=jnp.float32)
    mx = jnp.max(sim, axis=1, keepdims=True)
    li = jax.lax.broadcasted_iota(jnp.int32, (T, L), 1)
    labels = jnp.min(jnp.where(sim == mx, li, L), axis=1, keepdims=True)

    l0 = labels[0:M]
    l1 = labels[1:M + 1]
    l2 = labels[2:M + 2]
    l3 = labels[3:M + 3]

    def _cnt(a, c):
        return (a == c).astype(jnp.int32)

    c0 = 1 + _cnt(l0, l1) + _cnt(l0, l2) + _cnt(l0, l3)
    c1 = 1 + _cnt(l1, l0) + _cnt(l1, l2) + _cnt(l1, l3)
    c2 = 1 + _cnt(l2, l0) + _cnt(l2, l1) + _cnt(l2, l3)
    c3 = 1 + _cnt(l3, l0) + _cnt(l3, l1) + _cnt(l3, l2)
    s = jnp.maximum(
        jnp.maximum(c0 * 32 - l0, c1 * 32 - l1),
        jnp.maximum(c2 * 32 - l2, c3 * 32 - l3))
    maj = (-s) & 31

    oh = (maj == jax.lax.broadcasted_iota(jnp.int32, (M, 32), 1))
    e2p = jnp.concatenate(
        [e2, jnp.zeros((32 - L, C), jnp.float32)], axis=0)
    modrows = jnp.dot(oh.astype(jnp.float32), e2p,
                      preferred_element_type=jnp.float32)
    avg4 = (vis[0:M] + vis[1:M + 1] + vis[2:M + 2] + vis[3:M + 3]) * 0.25
    ma_ref[0, 0:M, :] = modrows * avg4
    ma_ref[0, M:T, :] = jnp.zeros((T - M, C), jnp.float32)


def _make_expand(B, T, NW, C):
    G = B * NW
    n_chunks = G * _K // _CHUNK
    n_workers = 32
    base = n_chunks // n_workers
    extra = n_chunks % n_workers
    gpc = _CHUNK // _K
    shift = 18
    magic = (1 << shift) // NW + 1
    assert all((g * magic) >> shift == g // NW for g in range(G))

    mesh = plsc.VectorSubcoreMesh(core_axis_name="c", subcore_axis_name="s")

    @functools.partial(
        pl.kernel,
        out_type=jax.ShapeDtypeStruct((G * _K, C), jnp.float32),
        mesh=mesh,
        scratch_types=[
            pltpu.VMEM((_NBUF, _CHUNK, C), jnp.float32),
            pltpu.VMEM((_NBUF, _CHUNK), jnp.int32),
            pltpu.SemaphoreType.DMA((_NBUF,)),
            pltpu.SemaphoreType.DMA((_NBUF,)),
        ],
    )
    def expand(tab_hbm, out_hbm, rows_v, idx_v, gsem, ssem):
        w = lax.axis_index("s") * 2 + lax.axis_index("c")
        my_n = base + jnp.where(w < extra, 1, 0).astype(jnp.int32)
        my_c0 = w * base + jnp.minimum(w, extra)
        lane = lax.iota(jnp.int32, 16)

        def idx_for(cc):
            g = cc * gpc + (lane >> 3)
            k = lane & 7
            b = (g * magic) >> shift
            nw = g - b * NW
            return b * T + nw + 4 * k

        def round_body(t, carry):
            i0 = t * _NBUF
            for j in range(_NBUF):
                i = i0 + j

                @pl.when(i < my_n)
                def _gather(i=i, j=j):
                    @pl.when(t >= 1)
                    def _wait_store():
                        pltpu.make_async_copy(
                            rows_v.at[j],
                            out_hbm.at[pl.ds(0, _CHUNK)],
                            ssem.at[j]).wait()
                    idx_v[j, :] = idx_for(my_c0 + i)
                    pltpu.async_copy(
                        tab_hbm.at[idx_v.at[j]],
                        rows_v.at[j], gsem.at[j])

            for j in range(_NBUF):
                i = i0 + j

                @pl.when(i < my_n)
                def _store(i=i, j=j):
                    cc = my_c0 + i
                    pltpu.make_async_copy(
                        tab_hbm.at[idx_v.at[j]],
                        rows_v.at[j], gsem.at[j]).wait()
                    pltpu.async_copy(
                        rows_v.at[j],
                        out_hbm.at[pl.ds(cc * _CHUNK, _CHUNK)],
                        ssem.at[j])
            return carry

        n_rounds = (my_n + _NBUF - 1) // _NBUF
        lax.fori_loop(0, n_rounds, round_body, 0)

        for j in range(_NBUF):
            @pl.when(my_n > j)
            def _drain(j=j):
                pltpu.make_async_copy(
                    rows_v.at[j],
                    out_hbm.at[pl.ds(0, _CHUNK)],
                    ssem.at[j]).wait()

    return expand


def kernel(vis_feats, query, W1, W2):
    B, T, C = vis_feats.shape
    L = query.shape[1]
    NW = T - _WIN + 1
    ma = pl.pallas_call(
        _modavg_kernel,
        grid=(B,),
        in_specs=[
            pl.BlockSpec((1, T, C), lambda b: (b, 0, 0)),
            pl.BlockSpec((1, L, C), lambda b: (b, 0, 0)),
            pl.BlockSpec((C, C), lambda b: (0, 0)),
            pl.BlockSpec((C, C), lambda b: (0, 0)),
        ],
        out_specs=pl.BlockSpec((1, T, C), lambda b: (b, 0, 0)),
        out_shape=jax.ShapeDtypeStruct((B, T, C), jnp.float32),
    )(vis_feats, query, W1, W2)
    expand = _make_expand(B, T, NW, C)
    out = expand(ma.reshape(B * T, C))
    return out.reshape(B, NW, _K, C)


# speedup vs baseline: 1.0354x; 1.0354x over previous
import functools
import jax
import jax.numpy as jnp
from jax import lax
from jax.experimental import pallas as pl
from jax.experimental.pallas import tpu as pltpu
from jax.experimental.pallas import tpu_sc as plsc

_WIN = 32
_K = 8
_NBUF = 3
_CHUNK = 64


def _modavg_kernel(vis_ref, q_ref, w1_ref, w2_ref, ma_ref):
    T = vis_ref.shape[1]
    C = vis_ref.shape[2]
    L = q_ref.shape[1]
    M = T - 3

    vis = vis_ref[0]
    q = q_ref[0]
    e1 = jnp.dot(q, w1_ref[...], preferred_element_type=jnp.float32)
    e2 = jnp.dot(q, w2_ref[...], preferred_element_type=jnp.float32)
    sim = jax.lax.dot_general(
        vis, e1, (((1,), (1,)), ((), ())),
        preferred_element_type=jnp.float32)
    mx = jnp.max(sim, axis=1, keepdims=True)
    li = jax.lax.broadcasted_iota(jnp.int32, (T, L), 1)
    labels = jnp.min(jnp.where(sim == mx, li, L), axis=1, keepdims=True)

    l0 = labels[0:M]
    l1 = labels[1:M + 1]
    l2 = labels[2:M + 2]
    l3 = labels[3:M + 3]

    def _cnt(a, c):
        return (a == c).astype(jnp.int32)

    c0 = 1 + _cnt(l0, l1) + _cnt(l0, l2) + _cnt(l0, l3)
    c1 = 1 + _cnt(l1, l0) + _cnt(l1, l2) + _cnt(l1, l3)
    c2 = 1 + _cnt(l2, l0) + _cnt(l2, l1) + _cnt(l2, l3)
    c3 = 1 + _cnt(l3, l0) + _cnt(l3, l1) + _cnt(l3, l2)
    s = jnp.maximum(
        jnp.maximum(c0 * 32 - l0, c1 * 32 - l1),
        jnp.maximum(c2 * 32 - l2, c3 * 32 - l3))
    maj = (-s) & 31

    oh = (maj == jax.lax.broadcasted_iota(jnp.int32, (M, 32), 1))
    e2p = jnp.concatenate(
        [e2, jnp.zeros((32 - L, C), jnp.float32)], axis=0)
    modrows = jnp.dot(oh.astype(jnp.float32), e2p,
                      preferred_element_type=jnp.float32)
    avg4 = (vis[0:M] + vis[1:M + 1] + vis[2:M + 2] + vis[3:M + 3]) * 0.25
    ma_ref[0:M, :] = modrows * avg4
    ma_ref[M:T, :] = jnp.zeros((T - M, C), jnp.float32)


def _make_expand(B, T, NW, C):
    G = B * NW
    n_chunks = G * _K // _CHUNK
    n_workers = 32
    base = n_chunks // n_workers
    extra = n_chunks % n_workers
    gpc = _CHUNK // _K
    shift = 18
    magic = (1 << shift) // NW + 1
    assert all((g * magic) >> shift == g // NW for g in range(G))

    mesh = plsc.VectorSubcoreMesh(core_axis_name="c", subcore_axis_name="s")

    @functools.partial(
        pl.kernel,
        out_type=jax.ShapeDtypeStruct((G * _K, C), jnp.float32),
        mesh=mesh,
        scratch_types=[
            pltpu.VMEM((_NBUF, _CHUNK, C), jnp.float32),
            pltpu.VMEM((_NBUF, _CHUNK), jnp.int32),
            pltpu.SemaphoreType.DMA((_NBUF,)),
            pltpu.SemaphoreType.DMA((_NBUF,)),
        ],
    )
    def expand(tab_hbm, out_hbm, rows_v, idx_v, gsem, ssem):
        w = lax.axis_index("s") * 2 + lax.axis_index("c")
        my_n = base + jnp.where(w < extra, 1, 0).astype(jnp.int32)
        my_c0 = w * base + jnp.minimum(w, extra)
        lane = lax.iota(jnp.int32, 16)

        def idx_for(cc, q):
            g = cc * gpc + q * 2 + (lane >> 3)
            k = lane & 7
            b = (g * magic) >> shift
            nw = g - b * NW
            return b * T + nw + 4 * k

        def round_body(t, carry):
            i0 = t * _NBUF
            for j in range(_NBUF):
                i = i0 + j

                @pl.when(i < my_n)
                def _gather(i=i, j=j):
                    @pl.when(t >= 1)
                    def _wait_store():
                        pltpu.make_async_copy(
                            rows_v.at[j],
                            out_hbm.at[pl.ds(0, _CHUNK)],
                            ssem.at[j]).wait()
                    for q in range(_CHUNK // 16):
                        idx_v[j, q * 16:(q + 1) * 16] = idx_for(my_c0 + i, q)
                    pltpu.async_copy(
                        tab_hbm.at[idx_v.at[j]],
                        rows_v.at[j], gsem.at[j])

            for j in range(_NBUF):
                i = i0 + j

                @pl.when(i < my_n)
                def _store(i=i, j=j):
                    cc = my_c0 + i
                    pltpu.make_async_copy(
                        tab_hbm.at[idx_v.at[j]],
                        rows_v.at[j], gsem.at[j]).wait()
                    pltpu.async_copy(
                        rows_v.at[j],
                        out_hbm.at[pl.ds(cc * _CHUNK, _CHUNK)],
                        ssem.at[j])
            return carry

        n_rounds = (my_n + _NBUF - 1) // _NBUF
        lax.fori_loop(0, n_rounds, round_body, 0)

        for j in range(_NBUF):
            @pl.when(my_n > j)
            def _drain(j=j):
                pltpu.make_async_copy(
                    rows_v.at[j],
                    out_hbm.at[pl.ds(0, _CHUNK)],
                    ssem.at[j]).wait()

    return expand


def kernel(vis_feats, query, W1, W2):
    B, T, C = vis_feats.shape
    L = query.shape[1]
    NW = T - _WIN + 1
    ma = pl.pallas_call(
        _modavg_kernel,
        grid=(B,),
        in_specs=[
            pl.BlockSpec((1, T, C), lambda b: (b, 0, 0)),
            pl.BlockSpec((1, L, C), lambda b: (b, 0, 0)),
            pl.BlockSpec((C, C), lambda b: (0, 0)),
            pl.BlockSpec((C, C), lambda b: (0, 0)),
        ],
        out_specs=pl.BlockSpec((T, C), lambda b: (b, 0)),
        out_shape=jax.ShapeDtypeStruct((B * T, C), jnp.float32),
    )(vis_feats, query, W1, W2)
    expand = _make_expand(B, T, NW, C)
    out = expand(ma)
    return out.reshape(B, NW, _K, C)


# speedup vs baseline: 1.0555x; 1.0194x over previous
import functools
import jax
import jax.numpy as jnp
from jax import lax
from jax.experimental import pallas as pl
from jax.experimental.pallas import tpu as pltpu
from jax.experimental.pallas import tpu_sc as plsc

_WIN = 32
_K = 8
_NBUF = 6
_CHUNK = 64


def _modavg_kernel(vis_ref, q_ref, w1_ref, w2_ref, ma_ref):
    T = vis_ref.shape[1]
    C = vis_ref.shape[2]
    L = q_ref.shape[1]
    M = T - 3

    vis = vis_ref[0]
    q = q_ref[0]
    e1 = jnp.dot(q, w1_ref[...], preferred_element_type=jnp.float32)
    e2 = jnp.dot(q, w2_ref[...], preferred_element_type=jnp.float32)
    sim = jax.lax.dot_general(
        vis, e1, (((1,), (1,)), ((), ())),
        preferred_element_type=jnp.float32)
    mx = jnp.max(sim, axis=1, keepdims=True)
    li = jax.lax.broadcasted_iota(jnp.int32, (T, L), 1)
    labels = jnp.min(jnp.where(sim == mx, li, L), axis=1, keepdims=True)

    l0 = labels[0:M]
    l1 = labels[1:M + 1]
    l2 = labels[2:M + 2]
    l3 = labels[3:M + 3]

    def _cnt(a, c):
        return (a == c).astype(jnp.int32)

    c0 = 1 + _cnt(l0, l1) + _cnt(l0, l2) + _cnt(l0, l3)
    c1 = 1 + _cnt(l1, l0) + _cnt(l1, l2) + _cnt(l1, l3)
    c2 = 1 + _cnt(l2, l0) + _cnt(l2, l1) + _cnt(l2, l3)
    c3 = 1 + _cnt(l3, l0) + _cnt(l3, l1) + _cnt(l3, l2)
    s = jnp.maximum(
        jnp.maximum(c0 * 32 - l0, c1 * 32 - l1),
        jnp.maximum(c2 * 32 - l2, c3 * 32 - l3))
    maj = (-s) & 31

    oh = (maj == jax.lax.broadcasted_iota(jnp.int32, (M, 32), 1))
    e2p = jnp.concatenate(
        [e2, jnp.zeros((32 - L, C), jnp.float32)], axis=0)
    modrows = jnp.dot(oh.astype(jnp.float32), e2p,
                      preferred_element_type=jnp.float32)
    avg4 = (vis[0:M] + vis[1:M + 1] + vis[2:M + 2] + vis[3:M + 3]) * 0.25
    ma_ref[0:M, :] = modrows * avg4
    ma_ref[M:T, :] = jnp.zeros((T - M, C), jnp.float32)


def _make_expand(B, T, NW, C):
    G = B * NW
    n_chunks = G * _K // _CHUNK
    n_workers = 32
    base = n_chunks // n_workers
    extra = n_chunks % n_workers
    gpc = _CHUNK // _K
    shift = 18
    magic = (1 << shift) // NW + 1
    assert all((g * magic) >> shift == g // NW for g in range(G))

    mesh = plsc.VectorSubcoreMesh(core_axis_name="c", subcore_axis_name="s")

    @functools.partial(
        pl.kernel,
        out_type=jax.ShapeDtypeStruct((G * _K, C), jnp.float32),
        mesh=mesh,
        scratch_types=[
            pltpu.VMEM((_NBUF, _CHUNK, C), jnp.float32),
            pltpu.VMEM((_NBUF, _CHUNK), jnp.int32),
            pltpu.SemaphoreType.DMA((_NBUF,)),
            pltpu.SemaphoreType.DMA((_NBUF,)),
        ],
    )
    def expand(tab_hbm, out_hbm, rows_v, idx_v, gsem, ssem):
        w = lax.axis_index("s") * 2 + lax.axis_index("c")
        my_n = base + jnp.where(w < extra, 1, 0).astype(jnp.int32)
        my_c0 = w * base + jnp.minimum(w, extra)
        lane = lax.iota(jnp.int32, 16)

        def idx_for(cc, q):
            g = cc * gpc + q * 2 + (lane >> 3)
            k = lane & 7
            b = (g * magic) >> shift
            nw = g - b * NW
            return b * T + nw + 4 * k

        def round_body(t, carry):
            i0 = t * _NBUF
            for j in range(_NBUF):
                i = i0 + j

                @pl.when(i < my_n)
                def _gather(i=i, j=j):
                    @pl.when(t >= 1)
                    def _wait_store():
                        pltpu.make_async_copy(
                            rows_v.at[j],
                            out_hbm.at[pl.ds(0, _CHUNK)],
                            ssem.at[j]).wait()
                    for q in range(_CHUNK // 16):
                        idx_v[j, q * 16:(q + 1) * 16] = idx_for(my_c0 + i, q)
                    pltpu.async_copy(
                        tab_hbm.at[idx_v.at[j]],
                        rows_v.at[j], gsem.at[j])

            for j in range(_NBUF):
                i = i0 + j

                @pl.when(i < my_n)
                def _store(i=i, j=j):
                    cc = my_c0 + i
                    pltpu.make_async_copy(
                        tab_hbm.at[idx_v.at[j]],
                        rows_v.at[j], gsem.at[j]).wait()
                    pltpu.async_copy(
                        rows_v.at[j],
                        out_hbm.at[pl.ds(cc * _CHUNK, _CHUNK)],
                        ssem.at[j])
            return carry

        n_rounds = (my_n + _NBUF - 1) // _NBUF
        lax.fori_loop(0, n_rounds, round_body, 0)

        for j in range(_NBUF):
            @pl.when(my_n > j)
            def _drain(j=j):
                pltpu.make_async_copy(
                    rows_v.at[j],
                    out_hbm.at[pl.ds(0, _CHUNK)],
                    ssem.at[j]).wait()

    return expand


def kernel(vis_feats, query, W1, W2):
    B, T, C = vis_feats.shape
    L = query.shape[1]
    NW = T - _WIN + 1
    ma = pl.pallas_call(
        _modavg_kernel,
        grid=(B,),
        in_specs=[
            pl.BlockSpec((1, T, C), lambda b: (b, 0, 0)),
            pl.BlockSpec((1, L, C), lambda b: (b, 0, 0)),
            pl.BlockSpec((C, C), lambda b: (0, 0)),
            pl.BlockSpec((C, C), lambda b: (0, 0)),
        ],
        out_specs=pl.BlockSpec((T, C), lambda b: (b, 0)),
        out_shape=jax.ShapeDtypeStruct((B * T, C), jnp.float32),
    )(vis_feats, query, W1, W2)
    expand = _make_expand(B, T, NW, C)
    out = expand(ma)
    return out.reshape(B, NW, _K, C)


# speedup vs baseline: 1.0613x; 1.0055x over previous
import functools
import jax
import jax.numpy as jnp
from jax import lax
from jax.experimental import pallas as pl
from jax.experimental.pallas import tpu as pltpu
from jax.experimental.pallas import tpu_sc as plsc

_WIN = 32
_K = 8
_NBUF = 7
_CHUNK = 64


def _modavg_kernel(vis_ref, q_ref, w1_ref, w2_ref, ma_ref):
    T = vis_ref.shape[1]
    C = vis_ref.shape[2]
    L = q_ref.shape[1]
    M = T - 3

    vis = vis_ref[0]
    q = q_ref[0]
    e1 = jnp.dot(q, w1_ref[...], preferred_element_type=jnp.float32)
    e2 = jnp.dot(q, w2_ref[...], preferred_element_type=jnp.float32)
    sim = jax.lax.dot_general(
        vis, e1, (((1,), (1,)), ((), ())),
        preferred_element_type=jnp.float32)
    mx = jnp.max(sim, axis=1, keepdims=True)
    li = jax.lax.broadcasted_iota(jnp.int32, (T, L), 1)
    labels = jnp.min(jnp.where(sim == mx, li, L), axis=1, keepdims=True)

    l0 = labels[0:M]
    l1 = labels[1:M + 1]
    l2 = labels[2:M + 2]
    l3 = labels[3:M + 3]

    def _cnt(a, c):
        return (a == c).astype(jnp.int32)

    c0 = 1 + _cnt(l0, l1) + _cnt(l0, l2) + _cnt(l0, l3)
    c1 = 1 + _cnt(l1, l0) + _cnt(l1, l2) + _cnt(l1, l3)
    c2 = 1 + _cnt(l2, l0) + _cnt(l2, l1) + _cnt(l2, l3)
    c3 = 1 + _cnt(l3, l0) + _cnt(l3, l1) + _cnt(l3, l2)
    s = jnp.maximum(
        jnp.maximum(c0 * 32 - l0, c1 * 32 - l1),
        jnp.maximum(c2 * 32 - l2, c3 * 32 - l3))
    maj = (-s) & 31

    oh = (maj == jax.lax.broadcasted_iota(jnp.int32, (M, 32), 1))
    e2p = jnp.concatenate(
        [e2, jnp.zeros((32 - L, C), jnp.float32)], axis=0)
    modrows = jnp.dot(oh.astype(jnp.float32), e2p,
                      preferred_element_type=jnp.float32)
    avg4 = (vis[0:M] + vis[1:M + 1] + vis[2:M + 2] + vis[3:M + 3]) * 0.25
    ma_ref[0:M, :] = modrows * avg4
    ma_ref[M:T, :] = jnp.zeros((T - M, C), jnp.float32)


def _make_expand(B, T, NW, C):
    G = B * NW
    n_chunks = G * _K // _CHUNK
    n_workers = 32
    base = n_chunks // n_workers
    extra = n_chunks % n_workers
    gpc = _CHUNK // _K
    shift = 18
    magic = (1 << shift) // NW + 1
    assert all((g * magic) >> shift == g // NW for g in range(G))

    mesh = plsc.VectorSubcoreMesh(core_axis_name="c", subcore_axis_name="s")

    @functools.partial(
        pl.kernel,
        out_type=jax.ShapeDtypeStruct((G * _K, C), jnp.float32),
        mesh=mesh,
        scratch_types=[
            pltpu.VMEM((_NBUF, _CHUNK, C), jnp.float32),
            pltpu.VMEM((_NBUF, _CHUNK), jnp.int32),
            pltpu.SemaphoreType.DMA((_NBUF,)),
            pltpu.SemaphoreType.DMA((_NBUF,)),
        ],
    )
    def expand(tab_hbm, out_hbm, rows_v, idx_v, gsem, ssem):
        w = lax.axis_index("s") * 2 + lax.axis_index("c")
        my_n = base + jnp.where(w < extra, 1, 0).astype(jnp.int32)
        my_c0 = w * base + jnp.minimum(w, extra)
        lane = lax.iota(jnp.int32, 16)

        def idx_for(cc, q):
            g = cc * gpc + q * 2 + (lane >> 3)
            k = lane & 7
            b = (g * magic) >> shift
            nw = g - b * NW
            return b * T + nw + 4 * k

        def round_body(t, carry):
            i0 = t * _NBUF
            for j in range(_NBUF):
                i = i0 + j

                @pl.when(i < my_n)
                def _gather(i=i, j=j):
                    @pl.when(t >= 1)
                    def _wait_store():
                        pltpu.make_async_copy(
                            rows_v.at[j],
                            out_hbm.at[pl.ds(0, _CHUNK)],
                            ssem.at[j]).wait()
                    for q in range(_CHUNK // 16):
                        idx_v[j, q * 16:(q + 1) * 16] = idx_for(my_c0 + i, q)
                    pltpu.async_copy(
                        tab_hbm.at[idx_v.at[j]],
                        rows_v.at[j], gsem.at[j])

            for j in range(_NBUF):
                i = i0 + j

                @pl.when(i < my_n)
                def _store(i=i, j=j):
                    cc = my_c0 + i
                    pltpu.make_async_copy(
                        tab_hbm.at[idx_v.at[j]],
                        rows_v.at[j], gsem.at[j]).wait()
                    pltpu.async_copy(
                        rows_v.at[j],
                        out_hbm.at[pl.ds(cc * _CHUNK, _CHUNK)],
                        ssem.at[j])
            return carry

        n_rounds = (my_n + _NBUF - 1) // _NBUF
        lax.fori_loop(0, n_rounds, round_body, 0)

        for j in range(_NBUF):
            @pl.when(my_n > j)
            def _drain(j=j):
                pltpu.make_async_copy(
                    rows_v.at[j],
                    out_hbm.at[pl.ds(0, _CHUNK)],
                    ssem.at[j]).wait()

    return expand


def kernel(vis_feats, query, W1, W2):
    B, T, C = vis_feats.shape
    L = query.shape[1]
    NW = T - _WIN + 1
    ma = pl.pallas_call(
        _modavg_kernel,
        grid=(B,),
        in_specs=[
            pl.BlockSpec((1, T, C), lambda b: (b, 0, 0)),
            pl.BlockSpec((1, L, C), lambda b: (b, 0, 0)),
            pl.BlockSpec((C, C), lambda b: (0, 0)),
            pl.BlockSpec((C, C), lambda b: (0, 0)),
        ],
        out_specs=pl.BlockSpec((T, C), lambda b: (b, 0)),
        out_shape=jax.ShapeDtypeStruct((B * T, C), jnp.float32),
    )(vis_feats, query, W1, W2)
    expand = _make_expand(B, T, NW, C)
    out = expand(ma)
    return out.reshape(B, NW, _K, C)
